# baseline (speedup 1.0000x reference)
import functools

import jax
import jax.numpy as jnp
from jax import lax
from jax.experimental import pallas as pl
from jax.experimental.pallas import tpu as pltpu
from jax.experimental.pallas import tpu_sc as plsc

N_NODES = 10000
D = 128
N_EDGES = 320000

NUM_CORES = 2
NUM_SUBCORES = 16

ROWS_PER_TILE = 632
R = ROWS_PER_TILE * NUM_SUBCORES
TRASH = R - 1

CHUNK = 128
CHUNKS_PER_TILE = 160
EDGES_PER_TILE = CHUNKS_PER_TILE * CHUNK
E_PAD = EDGES_PER_TILE * NUM_SUBCORES
SLAB_ROWS = NUM_CORES * NUM_SUBCORES * CHUNKS_PER_TILE

FIRE = 16

_mesh = plsc.VectorSubcoreMesh(core_axis_name="c", subcore_axis_name="s")


def _fill(ref, n, value):
    def body(k, carry):
        ref[pl.ds(16 * k, 16)] = jnp.full((16,), value, jnp.float32)
        return carry
    lax.fori_loop(0, n // 16, body, 0)


@functools.partial(
    pl.kernel,
    out_type=jax.ShapeDtypeStruct((NUM_CORES * 2 * R,), jnp.float32),
    mesh=_mesh,
    compiler_params=pltpu.CompilerParams(needs_layout_passes=False),
    scratch_types=[
        pltpu.VMEM((CHUNKS_PER_TILE // 4, CHUNK), jnp.int32),
        pltpu.VMEM((R,), jnp.float32),
        pltpu.VMEM((640,), jnp.float32),
        pltpu.VMEM((640,), jnp.float32),
        pltpu.VMEM_SHARED((NUM_SUBCORES * R,), jnp.float32),
    ],
)
def _degree_kernel(src_hbm, dst_hbm, deg_out,
                   idx_q, hist_l, tmp_v, acc_l, hist2d):
    c = lax.axis_index("c")
    s = lax.axis_index("s")
    slab0 = (c * NUM_SUBCORES + s) * CHUNKS_PER_TILE
    row0 = s * ROWS_PER_TILE
    nq = CHUNKS_PER_TILE // 4
    ones16 = jnp.ones((16,), jnp.float32)

    for phase, hbm in ((0, src_hbm), (1, dst_hbm)):
        _fill(hist_l, R, 0.0)
        for q in range(4):
            pltpu.sync_copy(hbm.at[pl.ds(slab0 + nq * q, nq), :], idx_q)

            def body(j, carry):
                for t in range(CHUNK // 16):
                    idx = idx_q[j, pl.ds(16 * t, 16)]
                    plsc.addupdate_scatter(hist_l, [idx], ones16)
                return carry

            lax.fori_loop(0, nq, body, 0)
        pltpu.sync_copy(hist_l, hist2d.at[pl.ds(s * R, R)])
        plsc.subcore_barrier()

        _fill(acc_l, 640, 0.0)
        for t in range(NUM_SUBCORES):
            pltpu.sync_copy(hist2d.at[pl.ds(t * R + row0, ROWS_PER_TILE)],
                            tmp_v.at[pl.ds(0, ROWS_PER_TILE)])

            def rbody(k, carry):
                sl = pl.ds(16 * k, 16)
                acc_l[sl] = acc_l[sl] + tmp_v[sl]
                return carry

            lax.fori_loop(0, 40, rbody, 0)
        pltpu.sync_copy(
            acc_l.at[pl.ds(0, ROWS_PER_TILE)],
            deg_out.at[pl.ds((2 * c + phase) * R + row0, ROWS_PER_TILE)])
        plsc.subcore_barrier()


N_QUARTERS = 4
Q_CHUNKS = CHUNKS_PER_TILE // N_QUARTERS


@functools.partial(
    pl.kernel,
    out_type=jax.ShapeDtypeStruct((NUM_CORES, R, D), jnp.float32),
    mesh=_mesh,
    scratch_types=[
        pltpu.VMEM((2 * Q_CHUNKS, CHUNK), jnp.int32),
        pltpu.VMEM((CHUNK, D), jnp.float32),
        pltpu.VMEM((CHUNK, D), jnp.float32),
        pltpu.VMEM((CHUNK,), jnp.int32),
        pltpu.VMEM((CHUNK,), jnp.int32),
        pltpu.VMEM((CHUNK,), jnp.int32),
        pltpu.VMEM((CHUNK,), jnp.int32),
        pltpu.VMEM_SHARED((R, D), jnp.float32),
        pltpu.SemaphoreType.DMA,
        pltpu.SemaphoreType.DMA,
    ],
)
def _agg_kernel(featg_hbm, comb_hbm, agg_out,
                comb_q, rows0, rows1, sidx0, sidx1, didx0, didx1,
                acc, sem0, sem1):
    c = lax.axis_index("c")
    s = lax.axis_index("s")
    slab0 = (c * NUM_SUBCORES + s) * CHUNKS_PER_TILE * 2
    row0 = s * ROWS_PER_TILE
    tail = ROWS_PER_TILE - 4 * CHUNK

    def zbody(k, carry):
        rows0[k // 8, pl.ds(16 * (k % 8), 16)] = jnp.zeros((16,), jnp.float32)
        return carry

    lax.fori_loop(0, CHUNK * (D // 16), zbody, 0)

    for k in range(4):
        pltpu.sync_copy(rows0, acc.at[pl.ds(row0 + CHUNK * k, CHUNK), :])
    pltpu.sync_copy(rows0.at[pl.ds(0, tail)],
                    acc.at[pl.ds(row0 + 4 * CHUNK, tail), :])
    plsc.subcore_barrier()

    for q in range(N_QUARTERS):
        pltpu.sync_copy(
            comb_hbm.at[pl.ds(slab0 + 2 * Q_CHUNKS * q, 2 * Q_CHUNKS), :],
            comb_q)

        def copyrow(dst_ref, srow):
            for t in range(CHUNK // 16):
                dst_ref[pl.ds(16 * t, 16)] = comb_q[srow, pl.ds(16 * t, 16)]

        def body(g, carry):
            copyrow(sidx0, 4 * g)
            copyrow(didx0, 4 * g + 1)
            copyrow(sidx1, 4 * g + 2)
            copyrow(didx1, 4 * g + 3)
            h0 = pltpu.async_copy(featg_hbm.at[sidx0], rows0, sem0)
            h1 = pltpu.async_copy(featg_hbm.at[sidx1], rows1, sem1)
            h0.wait()
            pltpu.sync_copy(rows0, acc.at[didx0], add=True)
            h1.wait()
            pltpu.sync_copy(rows1, acc.at[didx1], add=True)
            return carry

        lax.fori_loop(0, Q_CHUNKS // 2, body, 0)
    plsc.subcore_barrier()

    for k in range(4):
        pltpu.sync_copy(acc.at[pl.ds(row0 + CHUNK * k, CHUNK), :], rows0)
        pltpu.sync_copy(rows0, agg_out.at[c, pl.ds(row0 + CHUNK * k, CHUNK), :])
    pltpu.sync_copy(acc.at[pl.ds(row0 + 4 * CHUNK, tail), :],
                    rows0.at[pl.ds(0, tail)])
    pltpu.sync_copy(rows0.at[pl.ds(0, tail)],
                    agg_out.at[c, pl.ds(row0 + 4 * CHUNK, tail), :])


def _pack_edges(e0, e1):
    src2d = jnp.stack([e0[0], e1[0] + R]).reshape(SLAB_ROWS, CHUNK)
    dst2d = jnp.stack([e0[1], e1[1]]).reshape(SLAB_ROWS, CHUNK)
    return jnp.stack([src2d, dst2d], axis=1).reshape(2 * SLAB_ROWS, CHUNK)


def _scale_body(x_ref, od_ref, feat_ref):
    s_out = lax.rsqrt(jnp.maximum(od_ref[0], 1.0))
    feat_ref[...] = x_ref[...] * s_out


def _scale_call(x_pad, odg):
    nblk = R // ROWS_PER_TILE
    return pl.pallas_call(
        _scale_body,
        grid=(2, nblk),
        in_specs=[
            pl.BlockSpec((ROWS_PER_TILE, D), lambda h, i: (i, 0)),
            pl.BlockSpec((1, ROWS_PER_TILE, 1), lambda h, i: (h, i, 0)),
        ],
        out_specs=pl.BlockSpec((ROWS_PER_TILE, D), lambda h, i: (h * nblk + i, 0)),
        out_shape=jax.ShapeDtypeStruct((2 * R, D), jnp.float32),
    )(x_pad, odg)


def _out_body(a0_ref, a1_ref, i0_ref, i1_ref, w0_ref, w1_ref, b0_ref, b1_ref,
              y_ref):
    s0 = lax.rsqrt(jnp.maximum(i0_ref[...], 1.0))
    s1 = lax.rsqrt(jnp.maximum(i1_ref[...], 1.0))
    a0 = a0_ref[...] * s0
    a1 = a1_ref[...] * s1
    y = jnp.dot(a0, w0_ref[...], preferred_element_type=jnp.float32)
    y += jnp.dot(a1, w1_ref[...], preferred_element_type=jnp.float32)
    y_ref[...] = y + b0_ref[...] + b1_ref[...]


def _out_call(agg0, agg1, ind0, ind1, W0, W1, b0, b1):
    nblk = R // ROWS_PER_TILE
    return pl.pallas_call(
        _out_body,
        grid=(nblk,),
        in_specs=[
            pl.BlockSpec((ROWS_PER_TILE, D), lambda i: (i, 0)),
            pl.BlockSpec((ROWS_PER_TILE, D), lambda i: (i, 0)),
            pl.BlockSpec((ROWS_PER_TILE, 1), lambda i: (i, 0)),
            pl.BlockSpec((ROWS_PER_TILE, 1), lambda i: (i, 0)),
            pl.BlockSpec((D, D), lambda i: (0, 0)),
            pl.BlockSpec((D, D), lambda i: (0, 0)),
            pl.BlockSpec((1, D), lambda i: (0, 0)),
            pl.BlockSpec((1, D), lambda i: (0, 0)),
        ],
        out_specs=pl.BlockSpec((ROWS_PER_TILE, D), lambda i: (i, 0)),
        out_shape=jax.ShapeDtypeStruct((R, D), jnp.float32),
    )(agg0, agg1, ind0, ind1, W0, W1, b0, b1)


def kernel(x, edge_index_rel0, edge_index_rel1, W0, b0, W1, b1):
    e0 = edge_index_rel0.astype(jnp.int32)
    e1 = edge_index_rel1.astype(jnp.int32)
    pad = ((0, 0), (0, E_PAD - N_EDGES))
    e0 = jnp.pad(e0, pad, constant_values=TRASH)
    e1 = jnp.pad(e1, pad, constant_values=TRASH)
    src2d = jnp.stack([e0[0], e1[0]]).reshape(SLAB_ROWS, CHUNK)
    dst2d = jnp.stack([e0[1], e1[1]]).reshape(SLAB_ROWS, CHUNK)
    comb = _pack_edges(e0, e1)

    x_pad = jnp.pad(x, ((0, R - N_NODES), (0, 0)))

    degs = _degree_kernel(src2d, dst2d).reshape(2, 2, R)
    odg = degs[:, 0].reshape(2, R, 1)
    ind = degs[:, 1].reshape(2, R, 1)

    featg = _scale_call(x_pad, odg)

    agg = _agg_kernel(featg, comb)

    y = _out_call(agg[0], agg[1], ind[0], ind[1], W0, W1,
                  b0.reshape(1, D), b1.reshape(1, D))
    return y[:N_NODES]


# speedup vs baseline: 1.1230x; 1.1230x over previous
import functools

import jax
import jax.numpy as jnp
from jax import lax
from jax.experimental import pallas as pl
from jax.experimental.pallas import tpu as pltpu
from jax.experimental.pallas import tpu_sc as plsc

N_NODES = 10000
D = 128
N_EDGES = 320000

NUM_CORES = 2
NUM_SUBCORES = 16

ROWS_PER_TILE = 632
R = ROWS_PER_TILE * NUM_SUBCORES
TRASH = R - 1

CHUNK = 128
CHUNKS_PER_TILE = 160
EDGES_PER_TILE = CHUNKS_PER_TILE * CHUNK
E_PAD = EDGES_PER_TILE * NUM_SUBCORES
SLAB_ROWS = NUM_CORES * NUM_SUBCORES * CHUNKS_PER_TILE

FIRE = 16

_mesh = plsc.VectorSubcoreMesh(core_axis_name="c", subcore_axis_name="s")


def _fill(ref, n, value):
    def body(k, carry):
        ref[pl.ds(16 * k, 16)] = jnp.full((16,), value, jnp.float32)
        return carry
    lax.fori_loop(0, n // 16, body, 0)


@functools.partial(
    pl.kernel,
    out_type=jax.ShapeDtypeStruct((NUM_CORES * 2 * R,), jnp.float32),
    mesh=_mesh,
    compiler_params=pltpu.CompilerParams(needs_layout_passes=False),
    scratch_types=[
        pltpu.VMEM((CHUNKS_PER_TILE // 4, CHUNK), jnp.int32),
        pltpu.VMEM((R,), jnp.float32),
        pltpu.VMEM((640,), jnp.float32),
        pltpu.VMEM((640,), jnp.float32),
        pltpu.VMEM_SHARED((NUM_SUBCORES * R,), jnp.float32),
    ],
)
def _degree_kernel(src_hbm, dst_hbm, deg_out,
                   idx_q, hist_l, tmp_v, acc_l, hist2d):
    c = lax.axis_index("c")
    s = lax.axis_index("s")
    slab0 = (c * NUM_SUBCORES + s) * CHUNKS_PER_TILE
    row0 = s * ROWS_PER_TILE
    nq = CHUNKS_PER_TILE // 4
    ones16 = jnp.ones((16,), jnp.float32)

    for phase, hbm in ((0, src_hbm), (1, dst_hbm)):
        _fill(hist_l, R, 0.0)
        for q in range(4):
            pltpu.sync_copy(hbm.at[pl.ds(slab0 + nq * q, nq), :], idx_q)

            def body(j, carry):
                for t in range(CHUNK // 16):
                    idx = idx_q[j, pl.ds(16 * t, 16)]
                    plsc.addupdate_scatter(hist_l, [idx], ones16)
                return carry

            lax.fori_loop(0, nq, body, 0)
        pltpu.sync_copy(hist_l, hist2d.at[pl.ds(s * R, R)])
        plsc.subcore_barrier()

        _fill(acc_l, 640, 0.0)
        for t in range(NUM_SUBCORES):
            pltpu.sync_copy(hist2d.at[pl.ds(t * R + row0, ROWS_PER_TILE)],
                            tmp_v.at[pl.ds(0, ROWS_PER_TILE)])

            def rbody(k, carry):
                sl = pl.ds(16 * k, 16)
                acc_l[sl] = acc_l[sl] + tmp_v[sl]
                return carry

            lax.fori_loop(0, 40, rbody, 0)
        pltpu.sync_copy(
            acc_l.at[pl.ds(0, ROWS_PER_TILE)],
            deg_out.at[pl.ds((2 * c + phase) * R + row0, ROWS_PER_TILE)])
        plsc.subcore_barrier()


N_QUARTERS = 4
Q_CHUNKS = CHUNKS_PER_TILE // N_QUARTERS


@functools.partial(
    pl.kernel,
    out_type=jax.ShapeDtypeStruct((NUM_CORES, R, D), jnp.float32),
    mesh=_mesh,
    scratch_types=[
        pltpu.VMEM((2 * Q_CHUNKS, CHUNK), jnp.int32),
        pltpu.VMEM((CHUNK, D), jnp.float32),
        pltpu.VMEM((CHUNK, D), jnp.float32),
        pltpu.VMEM((CHUNK,), jnp.int32),
        pltpu.VMEM((CHUNK,), jnp.int32),
        pltpu.VMEM((CHUNK,), jnp.int32),
        pltpu.VMEM((CHUNK,), jnp.int32),
        pltpu.VMEM_SHARED((R, D), jnp.float32),
        pltpu.SemaphoreType.DMA,
        pltpu.SemaphoreType.DMA,
    ],
)
def _agg_kernel(featg_hbm, comb_hbm, agg_out,
                comb_q, rows0, rows1, sidx0, sidx1, didx0, didx1,
                acc, sem0, sem1):
    c = lax.axis_index("c")
    s = lax.axis_index("s")
    slab0 = (c * NUM_SUBCORES + s) * CHUNKS_PER_TILE * 2
    row0 = s * ROWS_PER_TILE
    tail = ROWS_PER_TILE - 4 * CHUNK

    def zbody(k, carry):
        rows0[k // 8, pl.ds(16 * (k % 8), 16)] = jnp.zeros((16,), jnp.float32)
        return carry

    lax.fori_loop(0, CHUNK * (D // 16), zbody, 0)

    for k in range(4):
        pltpu.sync_copy(rows0, acc.at[pl.ds(row0 + CHUNK * k, CHUNK), :])
    pltpu.sync_copy(rows0.at[pl.ds(0, tail)],
                    acc.at[pl.ds(row0 + 4 * CHUNK, tail), :])
    plsc.subcore_barrier()

    for q in range(N_QUARTERS):
        pltpu.sync_copy(
            comb_hbm.at[pl.ds(slab0 + 2 * Q_CHUNKS * q, 2 * Q_CHUNKS), :],
            comb_q)

        def copyrow(dst_ref, srow):
            for t in range(CHUNK // 16):
                dst_ref[pl.ds(16 * t, 16)] = comb_q[srow, pl.ds(16 * t, 16)]

        copyrow(sidx0, 0)
        copyrow(didx0, 1)
        copyrow(sidx1, 2)
        copyrow(didx1, 3)
        pltpu.async_copy(featg_hbm.at[sidx0], rows0, sem0)
        pltpu.async_copy(featg_hbm.at[sidx1], rows1, sem1)
        rows = (rows0, rows1)
        sems = (sem0, sem1)
        sidx = (sidx0, sidx1)
        didx = (didx0, didx1)

        def body(g, carry):
            for b in range(2):
                k = 2 * g + b
                pltpu.make_async_copy(
                    featg_hbm.at[pl.ds(0, CHUNK), :], rows[b], sems[b]).wait()
                pltpu.sync_copy(rows[b], acc.at[didx[b]], add=True)

                @pl.when(k + 2 < Q_CHUNKS)
                def _():
                    copyrow(sidx[b], 2 * (k + 2))
                    copyrow(didx[b], 2 * (k + 2) + 1)
                    pltpu.async_copy(featg_hbm.at[sidx[b]], rows[b], sems[b])
            return carry

        lax.fori_loop(0, Q_CHUNKS // 2, body, 0)
    plsc.subcore_barrier()

    for k in range(4):
        pltpu.sync_copy(acc.at[pl.ds(row0 + CHUNK * k, CHUNK), :], rows0)
        pltpu.sync_copy(rows0, agg_out.at[c, pl.ds(row0 + CHUNK * k, CHUNK), :])
    pltpu.sync_copy(acc.at[pl.ds(row0 + 4 * CHUNK, tail), :],
                    rows0.at[pl.ds(0, tail)])
    pltpu.sync_copy(rows0.at[pl.ds(0, tail)],
                    agg_out.at[c, pl.ds(row0 + 4 * CHUNK, tail), :])


def _pack_edges(e0, e1):
    src2d = jnp.stack([e0[0], e1[0] + R]).reshape(SLAB_ROWS, CHUNK)
    dst2d = jnp.stack([e0[1], e1[1]]).reshape(SLAB_ROWS, CHUNK)
    return jnp.stack([src2d, dst2d], axis=1).reshape(2 * SLAB_ROWS, CHUNK)


def _scale_body(x_ref, od_ref, feat_ref):
    s_out = lax.rsqrt(jnp.maximum(od_ref[0], 1.0))
    feat_ref[...] = x_ref[...] * s_out


def _scale_call(x_pad, odg):
    nblk = R // ROWS_PER_TILE
    return pl.pallas_call(
        _scale_body,
        grid=(2, nblk),
        in_specs=[
            pl.BlockSpec((ROWS_PER_TILE, D), lambda h, i: (i, 0)),
            pl.BlockSpec((1, ROWS_PER_TILE, 1), lambda h, i: (h, i, 0)),
        ],
        out_specs=pl.BlockSpec((ROWS_PER_TILE, D), lambda h, i: (h * nblk + i, 0)),
        out_shape=jax.ShapeDtypeStruct((2 * R, D), jnp.float32),
    )(x_pad, odg)


def _out_body(a0_ref, a1_ref, i0_ref, i1_ref, w0_ref, w1_ref, b0_ref, b1_ref,
              y_ref):
    s0 = lax.rsqrt(jnp.maximum(i0_ref[...], 1.0))
    s1 = lax.rsqrt(jnp.maximum(i1_ref[...], 1.0))
    a0 = a0_ref[...] * s0
    a1 = a1_ref[...] * s1
    y = jnp.dot(a0, w0_ref[...], preferred_element_type=jnp.float32)
    y += jnp.dot(a1, w1_ref[...], preferred_element_type=jnp.float32)
    y_ref[...] = y + b0_ref[...] + b1_ref[...]


def _out_call(agg0, agg1, ind0, ind1, W0, W1, b0, b1):
    nblk = R // ROWS_PER_TILE
    return pl.pallas_call(
        _out_body,
        grid=(nblk,),
        in_specs=[
            pl.BlockSpec((ROWS_PER_TILE, D), lambda i: (i, 0)),
            pl.BlockSpec((ROWS_PER_TILE, D), lambda i: (i, 0)),
            pl.BlockSpec((ROWS_PER_TILE, 1), lambda i: (i, 0)),
            pl.BlockSpec((ROWS_PER_TILE, 1), lambda i: (i, 0)),
            pl.BlockSpec((D, D), lambda i: (0, 0)),
            pl.BlockSpec((D, D), lambda i: (0, 0)),
            pl.BlockSpec((1, D), lambda i: (0, 0)),
            pl.BlockSpec((1, D), lambda i: (0, 0)),
        ],
        out_specs=pl.BlockSpec((ROWS_PER_TILE, D), lambda i: (i, 0)),
        out_shape=jax.ShapeDtypeStruct((R, D), jnp.float32),
    )(agg0, agg1, ind0, ind1, W0, W1, b0, b1)


def kernel(x, edge_index_rel0, edge_index_rel1, W0, b0, W1, b1):
    e0 = edge_index_rel0.astype(jnp.int32)
    e1 = edge_index_rel1.astype(jnp.int32)
    pad = ((0, 0), (0, E_PAD - N_EDGES))
    e0 = jnp.pad(e0, pad, constant_values=TRASH)
    e1 = jnp.pad(e1, pad, constant_values=TRASH)
    src2d = jnp.stack([e0[0], e1[0]]).reshape(SLAB_ROWS, CHUNK)
    dst2d = jnp.stack([e0[1], e1[1]]).reshape(SLAB_ROWS, CHUNK)
    comb = _pack_edges(e0, e1)

    x_pad = jnp.pad(x, ((0, R - N_NODES), (0, 0)))

    degs = _degree_kernel(src2d, dst2d).reshape(2, 2, R)
    odg = degs[:, 0].reshape(2, R, 1)
    ind = degs[:, 1].reshape(2, R, 1)

    featg = _scale_call(x_pad, odg)

    agg = _agg_kernel(featg, comb)

    y = _out_call(agg[0], agg[1], ind[0], ind[1], W0, W1,
                  b0.reshape(1, D), b1.reshape(1, D))
    return y[:N_NODES]


# speedup vs baseline: 1.1252x; 1.0020x over previous
import functools

import jax
import jax.numpy as jnp
from jax import lax
from jax.experimental import pallas as pl
from jax.experimental.pallas import tpu as pltpu
from jax.experimental.pallas import tpu_sc as plsc

N_NODES = 10000
D = 128
N_EDGES = 320000

NUM_CORES = 2
NUM_SUBCORES = 16

ROWS_PER_TILE = 632
R = ROWS_PER_TILE * NUM_SUBCORES
TRASH = R - 1

CHUNK = 128
CHUNKS_PER_TILE = 160
EDGES_PER_TILE = CHUNKS_PER_TILE * CHUNK
E_PAD = EDGES_PER_TILE * NUM_SUBCORES
SLAB_ROWS = NUM_CORES * NUM_SUBCORES * CHUNKS_PER_TILE

_mesh = plsc.VectorSubcoreMesh(core_axis_name="c", subcore_axis_name="s")


def _fill(ref, n, value):
    def body(k, carry):
        ref[pl.ds(16 * k, 16)] = jnp.full((16,), value, jnp.float32)
        return carry
    lax.fori_loop(0, n // 16, body, 0)


@functools.partial(
    pl.kernel,
    out_type=jax.ShapeDtypeStruct((NUM_CORES * 2 * R,), jnp.float32),
    mesh=_mesh,
    compiler_params=pltpu.CompilerParams(needs_layout_passes=False),
    scratch_types=[
        pltpu.VMEM((CHUNKS_PER_TILE // 4, CHUNK), jnp.int32),
        pltpu.VMEM((R,), jnp.float32),
        pltpu.VMEM((640,), jnp.float32),
        pltpu.VMEM((640,), jnp.float32),
        pltpu.VMEM_SHARED((NUM_SUBCORES * R,), jnp.float32),
    ],
)
def _degree_kernel(src_hbm, dst_hbm, deg_out,
                   idx_q, hist_l, tmp_v, acc_l, hist2d):
    c = lax.axis_index("c")
    s = lax.axis_index("s")
    slab0 = (c * NUM_SUBCORES + s) * CHUNKS_PER_TILE
    row0 = s * ROWS_PER_TILE
    nq = CHUNKS_PER_TILE // 4
    ones16 = jnp.ones((16,), jnp.float32)

    for phase, hbm in ((0, src_hbm), (1, dst_hbm)):
        _fill(hist_l, R, 0.0)
        for q in range(4):
            pltpu.sync_copy(hbm.at[pl.ds(slab0 + nq * q, nq), :], idx_q)

            def body(j, carry):
                for t in range(CHUNK // 16):
                    idx = idx_q[j, pl.ds(16 * t, 16)]
                    plsc.addupdate_scatter(hist_l, [idx], ones16)
                return carry

            lax.fori_loop(0, nq, body, 0)
        pltpu.sync_copy(hist_l, hist2d.at[pl.ds(s * R, R)])
        plsc.subcore_barrier()

        _fill(acc_l, 640, 0.0)
        for t in range(NUM_SUBCORES):
            pltpu.sync_copy(hist2d.at[pl.ds(t * R + row0, ROWS_PER_TILE)],
                            tmp_v.at[pl.ds(0, ROWS_PER_TILE)])

            def rbody(k, carry):
                sl = pl.ds(16 * k, 16)
                acc_l[sl] = acc_l[sl] + tmp_v[sl]
                return carry

            lax.fori_loop(0, 40, rbody, 0)
        pltpu.sync_copy(
            acc_l.at[pl.ds(0, ROWS_PER_TILE)],
            deg_out.at[pl.ds((2 * c + phase) * R + row0, ROWS_PER_TILE)])
        plsc.subcore_barrier()


N_QUARTERS = 4
Q_CHUNKS = CHUNKS_PER_TILE // N_QUARTERS


@functools.partial(
    pl.kernel,
    out_type=jax.ShapeDtypeStruct((NUM_CORES, R, D), jnp.float32),
    mesh=_mesh,
    scratch_types=[
        pltpu.VMEM((2 * Q_CHUNKS, CHUNK), jnp.int32),
        pltpu.VMEM((CHUNK, D), jnp.float32),
        pltpu.VMEM((CHUNK, D), jnp.float32),
        pltpu.VMEM((CHUNK,), jnp.int32),
        pltpu.VMEM((CHUNK,), jnp.int32),
        pltpu.VMEM((CHUNK,), jnp.int32),
        pltpu.VMEM((CHUNK,), jnp.int32),
        pltpu.VMEM_SHARED((R, D), jnp.float32),
        pltpu.SemaphoreType.DMA,
        pltpu.SemaphoreType.DMA,
    ],
)
def _agg_kernel(featg_hbm, comb_hbm, agg_out,
                comb_q, rows0, rows1, sidx0, sidx1, didx0, didx1,
                acc, sem0, sem1):
    c = lax.axis_index("c")
    s = lax.axis_index("s")
    slab0 = (c * NUM_SUBCORES + s) * CHUNKS_PER_TILE * 2
    row0 = s * ROWS_PER_TILE
    tail = ROWS_PER_TILE - 4 * CHUNK

    def zbody(k, carry):
        rows0[k // 8, pl.ds(16 * (k % 8), 16)] = jnp.zeros((16,), jnp.float32)
        return carry

    lax.fori_loop(0, CHUNK * (D // 16), zbody, 0)

    for k in range(4):
        pltpu.sync_copy(rows0, acc.at[pl.ds(row0 + CHUNK * k, CHUNK), :])
    pltpu.sync_copy(rows0.at[pl.ds(0, tail)],
                    acc.at[pl.ds(row0 + 4 * CHUNK, tail), :])
    plsc.subcore_barrier()

    for q in range(N_QUARTERS):
        pltpu.sync_copy(
            comb_hbm.at[pl.ds(slab0 + 2 * Q_CHUNKS * q, 2 * Q_CHUNKS), :],
            comb_q)

        def copyrow(dst_ref, srow):
            for t in range(CHUNK // 16):
                dst_ref[pl.ds(16 * t, 16)] = comb_q[srow, pl.ds(16 * t, 16)]

        copyrow(sidx0, 0)
        copyrow(didx0, 1)
        copyrow(sidx1, 2)
        copyrow(didx1, 3)
        pltpu.async_copy(featg_hbm.at[sidx0], rows0, sem0)
        pltpu.async_copy(featg_hbm.at[sidx1], rows1, sem1)
        rows = (rows0, rows1)
        sems = (sem0, sem1)
        sidx = (sidx0, sidx1)
        didx = (didx0, didx1)

        def body(g, carry):
            for b in range(2):
                k = 2 * g + b
                pltpu.make_async_copy(
                    featg_hbm.at[pl.ds(0, CHUNK), :], rows[b], sems[b]).wait()
                pltpu.sync_copy(rows[b], acc.at[didx[b]], add=True)

                @pl.when(k + 2 < Q_CHUNKS)
                def _():
                    copyrow(sidx[b], 2 * (k + 2))
                    copyrow(didx[b], 2 * (k + 2) + 1)
                    pltpu.async_copy(featg_hbm.at[sidx[b]], rows[b], sems[b])
            return carry

        lax.fori_loop(0, Q_CHUNKS // 2, body, 0)
    plsc.subcore_barrier()

    for k in range(4):
        pltpu.sync_copy(acc.at[pl.ds(row0 + CHUNK * k, CHUNK), :], rows0)
        pltpu.sync_copy(rows0, agg_out.at[c, pl.ds(row0 + CHUNK * k, CHUNK), :])
    pltpu.sync_copy(acc.at[pl.ds(row0 + 4 * CHUNK, tail), :],
                    rows0.at[pl.ds(0, tail)])
    pltpu.sync_copy(rows0.at[pl.ds(0, tail)],
                    agg_out.at[c, pl.ds(row0 + 4 * CHUNK, tail), :])


def _pack_edges(e0, e1):
    src2d = jnp.stack([e0[0], e1[0] + R]).reshape(SLAB_ROWS, CHUNK)
    dst2d = jnp.stack([e0[1], e1[1]]).reshape(SLAB_ROWS, CHUNK)
    return jnp.stack([src2d, dst2d], axis=1).reshape(2 * SLAB_ROWS, CHUNK)


def _scale_body(x_ref, od_ref, feat_ref):
    s_out = lax.rsqrt(jnp.maximum(od_ref[0], 1.0))
    feat_ref[...] = x_ref[...] * s_out


def _scale_call(x_pad, odg):
    nblk = R // ROWS_PER_TILE
    return pl.pallas_call(
        _scale_body,
        grid=(2, nblk),
        in_specs=[
            pl.BlockSpec((ROWS_PER_TILE, D), lambda h, i: (i, 0)),
            pl.BlockSpec((1, ROWS_PER_TILE, 1), lambda h, i: (h, i, 0)),
        ],
        out_specs=pl.BlockSpec((ROWS_PER_TILE, D), lambda h, i: (h * nblk + i, 0)),
        out_shape=jax.ShapeDtypeStruct((2 * R, D), jnp.float32),
    )(x_pad, odg)


def _out_body(a0_ref, a1_ref, i0_ref, i1_ref, w0_ref, w1_ref, b0_ref, b1_ref,
              y_ref):
    s0 = lax.rsqrt(jnp.maximum(i0_ref[...], 1.0))
    s1 = lax.rsqrt(jnp.maximum(i1_ref[...], 1.0))
    a0 = a0_ref[...] * s0
    a1 = a1_ref[...] * s1
    y = jnp.dot(a0, w0_ref[...], preferred_element_type=jnp.float32)
    y += jnp.dot(a1, w1_ref[...], preferred_element_type=jnp.float32)
    y_ref[...] = y + b0_ref[...] + b1_ref[...]


def _out_call(agg0, agg1, ind0, ind1, W0, W1, b0, b1):
    nblk = R // ROWS_PER_TILE
    return pl.pallas_call(
        _out_body,
        grid=(nblk,),
        in_specs=[
            pl.BlockSpec((ROWS_PER_TILE, D), lambda i: (i, 0)),
            pl.BlockSpec((ROWS_PER_TILE, D), lambda i: (i, 0)),
            pl.BlockSpec((ROWS_PER_TILE, 1), lambda i: (i, 0)),
            pl.BlockSpec((ROWS_PER_TILE, 1), lambda i: (i, 0)),
            pl.BlockSpec((D, D), lambda i: (0, 0)),
            pl.BlockSpec((D, D), lambda i: (0, 0)),
            pl.BlockSpec((1, D), lambda i: (0, 0)),
            pl.BlockSpec((1, D), lambda i: (0, 0)),
        ],
        out_specs=pl.BlockSpec((ROWS_PER_TILE, D), lambda i: (i, 0)),
        out_shape=jax.ShapeDtypeStruct((R, D), jnp.float32),
    )(agg0, agg1, ind0, ind1, W0, W1, b0, b1)


def kernel(x, edge_index_rel0, edge_index_rel1, W0, b0, W1, b1):
    e0 = edge_index_rel0.astype(jnp.int32)
    e1 = edge_index_rel1.astype(jnp.int32)
    pad = ((0, 0), (0, E_PAD - N_EDGES))
    e0 = jnp.pad(e0, pad, constant_values=TRASH)
    e1 = jnp.pad(e1, pad, constant_values=TRASH)
    src2d = jnp.stack([e0[0], e1[0]]).reshape(SLAB_ROWS, CHUNK)
    dst2d = jnp.stack([e0[1], e1[1]]).reshape(SLAB_ROWS, CHUNK)
    comb = _pack_edges(e0, e1)

    x_pad = jnp.pad(x, ((0, R - N_NODES), (0, 0)))

    degs = _degree_kernel(src2d, dst2d).reshape(2, 2, R)
    odg = degs[:, 0].reshape(2, R, 1)
    ind = degs[:, 1].reshape(2, R, 1)

    featg = _scale_call(x_pad, odg)

    agg = _agg_kernel(featg, comb)

    y = _out_call(agg[0], agg[1], ind[0], ind[1], W0, W1,
                  b0.reshape(1, D), b1.reshape(1, D))
    return y[:N_NODES]
